# baseline (speedup 1.0000x reference)
import jax
import jax.numpy as jnp
from jax import lax
from jax.experimental import pallas as pl
from jax.experimental.pallas import tpu as pltpu

B = 8
C = 256
K = 1024
P = 1024
N = B * P
_LOSS_SCALE = 0.75 / (N * C)


def _body(z_ref, cb_ref, zq_ref, idx_ref, loss_ref):
    b = pl.program_id(0)
    zb = z_ref[0]
    cb = cb_ref[...]

    dot2 = lax.dot_general(cb, zb + zb, (((1,), (0,)), ((), ())),
                           preferred_element_type=jnp.float32)
    z2 = jnp.sum(zb * zb, axis=0, keepdims=True)
    c2 = jnp.sum(cb * cb, axis=1, keepdims=True)
    a = z2 + c2
    dist = a - dot2

    minv = jnp.min(dist, axis=0, keepdims=True)
    iota = lax.broadcasted_iota(jnp.int32, (K, P), 0).astype(jnp.float32)
    idx_f = jnp.min(jnp.where(dist == minv, iota, float(K)),
                    axis=0, keepdims=True)
    idx_ref[0] = idx_f.astype(jnp.int32)

    onehot = jnp.where(iota == idx_f, 1.0, 0.0)
    zq = lax.dot_general(cb, onehot, (((0,), (0,)), ((), ())),
                         preferred_element_type=jnp.float32)
    zq_ref[0] = zq.reshape(C, 32, 32)

    part = jnp.sum(minv)
    @pl.when(b == 0)
    def _():
        loss_ref[0, 0] = part

    @pl.when(b > 0)
    def _():
        loss_ref[0, 0] = loss_ref[0, 0] + part

    @pl.when(b == B - 1)
    def _():
        loss_ref[0, 0] = loss_ref[0, 0] * _LOSS_SCALE


@jax.jit
def kernel(z, codebook):
    z3 = z.reshape(B, C, P)
    zq3, idx3, loss = pl.pallas_call(
        _body,
        grid=(B,),
        in_specs=[
            pl.BlockSpec((1, C, P), lambda b: (b, 0, 0)),
            pl.BlockSpec((K, C), lambda b: (0, 0)),
        ],
        out_specs=[
            pl.BlockSpec((1, C, 32, 32), lambda b: (b, 0, 0, 0)),
            pl.BlockSpec((1, 1, P), lambda b: (b, 0, 0)),
            pl.BlockSpec(memory_space=pltpu.SMEM),
        ],
        out_shape=[
            jax.ShapeDtypeStruct((B, C, 32, 32), jnp.float32),
            jax.ShapeDtypeStruct((B, 1, P), jnp.int32),
            jax.ShapeDtypeStruct((1, 1), jnp.float32),
        ],
        compiler_params=pltpu.CompilerParams(
            allow_input_fusion=[True, False],
        ),
    )(z3, codebook)
    return (zq3, idx3.reshape(N), loss[0, 0])


# speedup vs baseline: 1.5703x; 1.5703x over previous
import jax
import jax.numpy as jnp
from jax import lax
from jax.experimental import pallas as pl
from jax.experimental.pallas import tpu as pltpu

B = 8
C = 256
K = 1024
P = 1024
N = B * P
_LOSS_SCALE = 0.75 / (N * C)


BPP = 2


def _one_batch(zb, cb, c2, iota):
    dot2 = lax.dot_general(cb, zb + zb, (((1,), (0,)), ((), ())),
                           preferred_element_type=jnp.float32)
    z2 = jnp.sum(zb * zb, axis=0, keepdims=True)
    a = z2 + c2
    dist = a - dot2

    minv = jnp.min(dist, axis=0, keepdims=True)
    idx_f = jnp.min(jnp.where(dist == minv, iota, float(K)),
                    axis=0, keepdims=True)

    onehot = jnp.where(iota == idx_f, 1.0, 0.0)
    zq = lax.dot_general(cb, onehot, (((0,), (0,)), ((), ())),
                         preferred_element_type=jnp.float32)
    return zq, idx_f.astype(jnp.int32), jnp.sum(minv)


def _body(z_ref, cb_ref, zq_ref, idx_ref, loss_ref):
    g = pl.program_id(0)
    cb = cb_ref[...]
    c2 = jnp.sum(cb * cb, axis=1, keepdims=True)
    iota = lax.broadcasted_iota(jnp.int32, (K, P), 0).astype(jnp.float32)

    part = jnp.float32(0.0)
    for j in range(BPP):
        zq, idx_i, p_j = _one_batch(z_ref[j], cb, c2, iota)
        zq_ref[j] = zq
        idx_ref[j] = idx_i
        part = part + p_j

    @pl.when(g == 0)
    def _():
        loss_ref[0, 0] = part

    @pl.when(g > 0)
    def _():
        loss_ref[0, 0] = loss_ref[0, 0] + part

    @pl.when(g == (B // BPP) - 1)
    def _():
        loss_ref[0, 0] = loss_ref[0, 0] * _LOSS_SCALE


@jax.jit
def kernel(z, codebook):
    z3 = z.reshape(B, C, P)
    zq3, idx3, loss = pl.pallas_call(
        _body,
        grid=(B // BPP,),
        in_specs=[
            pl.BlockSpec((BPP, C, P), lambda g: (g, 0, 0)),
            pl.BlockSpec((K, C), lambda g: (0, 0)),
        ],
        out_specs=[
            pl.BlockSpec((BPP, C, P), lambda g: (g, 0, 0)),
            pl.BlockSpec((BPP, 1, P), lambda g: (g, 0, 0)),
            pl.BlockSpec(memory_space=pltpu.SMEM),
        ],
        out_shape=[
            jax.ShapeDtypeStruct((B, C, P), jnp.float32),
            jax.ShapeDtypeStruct((B, 1, P), jnp.int32),
            jax.ShapeDtypeStruct((1, 1), jnp.float32),
        ],
    )(z3, codebook)
    return (zq3.reshape(B, C, 32, 32), idx3.reshape(N), loss[0, 0])
